# baseline (speedup 1.0000x reference)
import functools

import jax
import jax.numpy as jnp
from jax import lax
from jax.experimental import pallas as pl
from jax.experimental.pallas import tpu as pltpu
from jax.experimental.pallas import tpu_sc as plsc

B = 16384
D = 16
V = 1000000
L = 16
NC = 2
NS = 16
NW = NC * NS
BPW = B // NW
CHUNK = 128
NCH = BPW // CHUNK
NGRP = BPW // L
TAILBASE = (V // 128) * 128
NTAIL = V - TAILBASE
CW = 62464
SUBW = 7808
LEFT0 = CW * NS
LEFTN = TAILBASE - LEFT0


def _detile_body(ut_hbm, pt_hbm, utd_out, ptd_out, zb, sem):
    wid = lax.axis_index("s") * NC + lax.axis_index("c")

    for rr in range(D):
        for c in range(NTAIL // L):
            zb[rr, pl.ds(c * L, L)] = jnp.zeros((L,), jnp.float32)

    for tbl, out in ((ut_hbm, utd_out), (pt_hbm, ptd_out)):
        for s in range(CW // SUBW):
            col = pl.multiple_of(wid * CW + s * SUBW, 128)
            pltpu.async_copy(
                tbl.at[:, pl.ds(col, SUBW)], out.at[:, pl.ds(col, SUBW)], sem
            ).wait()

        @pl.when(wid == 0)
        def _():
            pltpu.async_copy(
                tbl.at[:, pl.ds(LEFT0, LEFTN)],
                out.at[:, pl.ds(LEFT0, LEFTN)],
                sem,
            ).wait()
            pltpu.async_copy(zb, out.at[:, pl.ds(TAILBASE, NTAIL)], sem).wait()


@functools.cache
def _build_detile():
    mesh = plsc.VectorSubcoreMesh(
        core_axis_name="c", subcore_axis_name="s", num_cores=NC, num_subcores=NS
    )
    return pl.kernel(
        _detile_body,
        out_type=(
            jax.ShapeDtypeStruct((D, V), jnp.float32),
            jax.ShapeDtypeStruct((D, V), jnp.float32),
        ),
        mesh=mesh,
        scratch_types=[
            pltpu.VMEM((D, NTAIL), jnp.float32),
            pltpu.SemaphoreType.DMA,
        ],
        compiler_params=pltpu.CompilerParams(
            needs_layout_passes=False, use_tc_tiling_on_sc=True
        ),
    )


def _score_body(
    uid_hbm,
    pid_hbm,
    nid_hbm,
    utd_hbm,
    ptd_hbm,
    tu_hbm,
    tp_hbm,
    pos_out,
    neg_out,
    uid_v,
    pid_v,
    nid_v,
    u_val,
    i_val,
    j_val,
    ps_v,
    ns_v,
    tail_u,
    tail_p,
    sem,
):
    wid = lax.axis_index("s") * NC + lax.axis_index("c")
    base = wid * BPW

    pltpu.sync_copy(uid_hbm.at[pl.ds(base, BPW)], uid_v)
    pltpu.sync_copy(pid_hbm.at[pl.ds(base, BPW)], pid_v)
    pltpu.sync_copy(nid_hbm.at[pl.ds(base, BPW)], nid_v)
    pltpu.sync_copy(tu_hbm, tail_u)
    pltpu.sync_copy(tp_hbm, tail_p)

    zero = jnp.zeros((L,), jnp.float32)
    for g in range(NGRP):
        ps_v[pl.ds(g * L, L)] = zero
        ns_v[pl.ds(g * L, L)] = zero

    lane = lax.iota(jnp.int32, L)

    def any_tail(idx_ref):
        def body(v, flag):
            m = idx_ref[pl.ds(v * L, L)] >= TAILBASE
            return flag + jnp.max(plsc.all_reduce_population_count(m))

        return lax.fori_loop(0, NGRP, body, jnp.int32(0))

    t_u = any_tail(uid_v)
    t_p = any_tail(pid_v)
    t_n = any_tail(nid_v)

    def patch(idx_ref, val_ref, tail_ref, d):
        def body(v, carry):
            ids = idx_ref[pl.ds(v * L, L)]
            m = ids >= TAILBASE
            tv = plsc.load_gather(tail_ref, [(ids - TAILBASE) * D + d], mask=m)
            plsc.store_scatter(val_ref, [v * L + lane], tv, mask=m)
            return carry

        lax.fori_loop(0, NGRP, body, 0)

    for d in range(D):
        copies = []
        for k in range(NCH):
            dst = pl.ds(k * CHUNK, CHUNK)
            copies.append(
                pltpu.async_copy(
                    utd_hbm.at[d].at[uid_v.at[pl.ds(k * CHUNK, CHUNK)]],
                    u_val.at[dst],
                    sem,
                )
            )
            copies.append(
                pltpu.async_copy(
                    ptd_hbm.at[d].at[pid_v.at[pl.ds(k * CHUNK, CHUNK)]],
                    i_val.at[dst],
                    sem,
                )
            )
            copies.append(
                pltpu.async_copy(
                    ptd_hbm.at[d].at[nid_v.at[pl.ds(k * CHUNK, CHUNK)]],
                    j_val.at[dst],
                    sem,
                )
            )
        for cp in copies:
            cp.wait()

        @pl.when(t_u > 0)
        def _():
            patch(uid_v, u_val, tail_u, d)

        @pl.when(t_p > 0)
        def _():
            patch(pid_v, i_val, tail_p, d)

        @pl.when(t_n > 0)
        def _():
            patch(nid_v, j_val, tail_p, d)

        def acc(g, carry):
            s = pl.ds(g * L, L)
            uv = u_val[s]
            ps_v[s] = ps_v[s] + uv * i_val[s]
            ns_v[s] = ns_v[s] + uv * j_val[s]
            return carry

        lax.fori_loop(0, NGRP, acc, 0)

    pltpu.sync_copy(ps_v, pos_out.at[pl.ds(base, BPW)])
    pltpu.sync_copy(ns_v, neg_out.at[pl.ds(base, BPW)])


@functools.cache
def _build_score():
    mesh = plsc.VectorSubcoreMesh(
        core_axis_name="c", subcore_axis_name="s", num_cores=NC, num_subcores=NS
    )
    return pl.kernel(
        _score_body,
        out_type=(
            jax.ShapeDtypeStruct((B,), jnp.float32),
            jax.ShapeDtypeStruct((B,), jnp.float32),
        ),
        mesh=mesh,
        scratch_types=[
            pltpu.VMEM((BPW,), jnp.int32),
            pltpu.VMEM((BPW,), jnp.int32),
            pltpu.VMEM((BPW,), jnp.int32),
            pltpu.VMEM((BPW,), jnp.float32),
            pltpu.VMEM((BPW,), jnp.float32),
            pltpu.VMEM((BPW,), jnp.float32),
            pltpu.VMEM((BPW,), jnp.float32),
            pltpu.VMEM((BPW,), jnp.float32),
            pltpu.VMEM((NTAIL * D,), jnp.float32),
            pltpu.VMEM((NTAIL * D,), jnp.float32),
            pltpu.SemaphoreType.DMA,
        ],
        compiler_params=pltpu.CompilerParams(
            needs_layout_passes=False, use_tc_tiling_on_sc=False
        ),
    )


def kernel(user_ids, pos_pids, neg_pids, user_table, playlist_table):
    utd, ptd = _build_detile()(user_table.T, playlist_table.T)
    pos, neg = _build_score()(
        user_ids.astype(jnp.int32),
        pos_pids.astype(jnp.int32),
        neg_pids.astype(jnp.int32),
        utd,
        ptd,
        user_table[TAILBASE:].reshape(-1),
        playlist_table[TAILBASE:].reshape(-1),
    )
    return (pos, neg)


# speedup vs baseline: 1.6841x; 1.6841x over previous
import functools

import jax
import jax.numpy as jnp
from jax import lax
from jax.experimental import pallas as pl
from jax.experimental.pallas import tpu as pltpu
from jax.experimental.pallas import tpu_sc as plsc

B = 16384
D = 16
V = 1000000
L = 16
NC = 2
NS = 16
NW = NC * NS
BPW = B // NW
CHUNK = 128
NCH = BPW // CHUNK
NGRP = BPW // L
TAILBASE = (V // 128) * 128
NTAIL = V - TAILBASE
VP = 1000064
TRS = (VP // 128) * 1024
CWP = 31232
CWPL = TAILBASE - CWP * (NW - 1)


def _pack_body(ut_hbm, pt_hbm, uo, po, sem):
    wid = lax.axis_index("s") * NC + lax.axis_index("c")

    for tbl, out in ((ut_hbm, uo), (pt_hbm, po)):

        @pl.when(wid < NW - 1)
        def _():
            col = pl.multiple_of(wid * CWP, 128)
            pltpu.async_copy(
                tbl.at[:, pl.ds(col, CWP)], out.at[:, pl.ds(col, CWP)], sem
            ).wait()

        @pl.when(wid == NW - 1)
        def _():
            pltpu.async_copy(
                tbl.at[:, pl.ds((NW - 1) * CWP, CWPL)],
                out.at[:, pl.ds((NW - 1) * CWP, CWPL)],
                sem,
            ).wait()


@functools.cache
def _build_pack():
    mesh = plsc.VectorSubcoreMesh(
        core_axis_name="c", subcore_axis_name="s", num_cores=NC, num_subcores=NS
    )
    return pl.kernel(
        _pack_body,
        out_type=(
            jax.ShapeDtypeStruct((D, VP), jnp.float32),
            jax.ShapeDtypeStruct((D, VP), jnp.float32),
        ),
        mesh=mesh,
        scratch_types=[pltpu.SemaphoreType.DMA],
        compiler_params=pltpu.CompilerParams(
            needs_layout_passes=False, use_tc_tiling_on_sc=True
        ),
    )


def _score_body(
    uid_hbm,
    pid_hbm,
    nid_hbm,
    utf_hbm,
    ptf_hbm,
    tu_hbm,
    tp_hbm,
    pos_out,
    neg_out,
    uid_v,
    pid_v,
    nid_v,
    ub_u,
    ub_p,
    ub_n,
    ix_u,
    ix_p,
    ix_n,
    u_val,
    i_val,
    j_val,
    ps_v,
    ns_v,
    tail_u,
    tail_p,
    sem,
):
    wid = lax.axis_index("s") * NC + lax.axis_index("c")
    base = wid * BPW

    pltpu.sync_copy(uid_hbm.at[pl.ds(base, BPW)], uid_v)
    pltpu.sync_copy(pid_hbm.at[pl.ds(base, BPW)], pid_v)
    pltpu.sync_copy(nid_hbm.at[pl.ds(base, BPW)], nid_v)
    pltpu.sync_copy(tu_hbm, tail_u)
    pltpu.sync_copy(tp_hbm, tail_p)

    zero = jnp.zeros((L,), jnp.float32)
    for g in range(NGRP):
        ps_v[pl.ds(g * L, L)] = zero
        ns_v[pl.ds(g * L, L)] = zero

    lane = lax.iota(jnp.int32, L)

    def any_tail(idx_ref):
        def body(v, flag):
            m = idx_ref[pl.ds(v * L, L)] >= TAILBASE
            return flag + jnp.max(plsc.all_reduce_population_count(m))

        return lax.fori_loop(0, NGRP, body, jnp.int32(0))

    t_u = any_tail(uid_v)
    t_p = any_tail(pid_v)
    t_n = any_tail(nid_v)

    def mkbase(idx_ref, ub_ref):
        def body(v, carry):
            s2 = pl.ds(v * L, L)
            ub_ref[s2] = idx_ref[s2]
            return carry

        lax.fori_loop(0, NGRP, body, 0)

    mkbase(uid_v, ub_u)
    mkbase(pid_v, ub_p)
    mkbase(nid_v, ub_n)

    def patch(idx_ref, val_ref, tail_ref, d):
        def body(v, carry):
            ids = idx_ref[pl.ds(v * L, L)]
            m = ids >= TAILBASE
            tv = plsc.load_gather(tail_ref, [(ids - TAILBASE) * D + d], mask=m)
            plsc.store_scatter(val_ref, [v * L + lane], tv, mask=m)
            return carry

        lax.fori_loop(0, NGRP, body, 0)

    for d in range(D):
        doff = d * VP

        def mkidx(ub_ref, ix_ref, c2):
            def body(v, carry):
                s2 = pl.ds(v * L, L)
                ix_ref[s2] = ub_ref[s2] + doff
                return carry

            lax.fori_loop(0, NGRP, body, c2)

        mkidx(ub_u, ix_u, 0)
        mkidx(ub_p, ix_p, 0)
        mkidx(ub_n, ix_n, 0)

        copies = []
        for k in range(NCH):
            dst = pl.ds(k * CHUNK, CHUNK)
            copies.append(
                pltpu.async_copy(
                    utf_hbm.at[ix_u.at[pl.ds(k * CHUNK, CHUNK)]],
                    u_val.at[dst],
                    sem,
                )
            )
            copies.append(
                pltpu.async_copy(
                    ptf_hbm.at[ix_p.at[pl.ds(k * CHUNK, CHUNK)]],
                    i_val.at[dst],
                    sem,
                )
            )
            copies.append(
                pltpu.async_copy(
                    ptf_hbm.at[ix_n.at[pl.ds(k * CHUNK, CHUNK)]],
                    j_val.at[dst],
                    sem,
                )
            )
        for cp in copies:
            cp.wait()

        @pl.when(t_u > 0)
        def _():
            patch(uid_v, u_val, tail_u, d)

        @pl.when(t_p > 0)
        def _():
            patch(pid_v, i_val, tail_p, d)

        @pl.when(t_n > 0)
        def _():
            patch(nid_v, j_val, tail_p, d)

        def acc(g, carry):
            s = pl.ds(g * L, L)
            uv = u_val[s]
            ps_v[s] = ps_v[s] + uv * i_val[s]
            ns_v[s] = ns_v[s] + uv * j_val[s]
            return carry

        lax.fori_loop(0, NGRP, acc, 0)

    pltpu.sync_copy(ps_v, pos_out.at[pl.ds(base, BPW)])
    pltpu.sync_copy(ns_v, neg_out.at[pl.ds(base, BPW)])


@functools.cache
def _build_score():
    mesh = plsc.VectorSubcoreMesh(
        core_axis_name="c", subcore_axis_name="s", num_cores=NC, num_subcores=NS
    )
    return pl.kernel(
        _score_body,
        out_type=(
            jax.ShapeDtypeStruct((B,), jnp.float32),
            jax.ShapeDtypeStruct((B,), jnp.float32),
        ),
        mesh=mesh,
        scratch_types=[
            pltpu.VMEM((BPW,), jnp.int32),
            pltpu.VMEM((BPW,), jnp.int32),
            pltpu.VMEM((BPW,), jnp.int32),
            pltpu.VMEM((BPW,), jnp.int32),
            pltpu.VMEM((BPW,), jnp.int32),
            pltpu.VMEM((BPW,), jnp.int32),
            pltpu.VMEM((BPW,), jnp.int32),
            pltpu.VMEM((BPW,), jnp.int32),
            pltpu.VMEM((BPW,), jnp.int32),
            pltpu.VMEM((BPW,), jnp.float32),
            pltpu.VMEM((BPW,), jnp.float32),
            pltpu.VMEM((BPW,), jnp.float32),
            pltpu.VMEM((BPW,), jnp.float32),
            pltpu.VMEM((BPW,), jnp.float32),
            pltpu.VMEM((NTAIL * D,), jnp.float32),
            pltpu.VMEM((NTAIL * D,), jnp.float32),
            pltpu.SemaphoreType.DMA,
        ],
        compiler_params=pltpu.CompilerParams(
            needs_layout_passes=False, use_tc_tiling_on_sc=False
        ),
    )


def kernel(user_ids, pos_pids, neg_pids, user_table, playlist_table):
    utp, ptp = _build_pack()(user_table.T, playlist_table.T)
    pos, neg = _build_score()(
        user_ids.astype(jnp.int32),
        pos_pids.astype(jnp.int32),
        neg_pids.astype(jnp.int32),
        utp.reshape(-1),
        ptp.reshape(-1),
        user_table[TAILBASE:].reshape(-1),
        playlist_table[TAILBASE:].reshape(-1),
    )
    return (pos, neg)


# speedup vs baseline: 12.8298x; 7.6183x over previous
import functools

import jax
import jax.numpy as jnp
from jax import lax
from jax.experimental import pallas as pl
from jax.experimental.pallas import tpu as pltpu
from jax.experimental.pallas import tpu_sc as plsc

B = 16384
D = 16
L = 16
NC = 2
NS = 16
NW = NC * NS
BPW = B // NW
CHUNK = 128
NCH = BPW // CHUNK
NGRP = BPW // L


def _bpr_sc_body(
    uid_hbm,
    pid_hbm,
    nid_hbm,
    ut_hbm,
    pt_hbm,
    pos_out,
    neg_out,
    uid_v,
    pid_v,
    nid_v,
    u_rows,
    i_rows,
    j_rows,
    ps_v,
    ns_v,
    sem,
):
    wid = lax.axis_index("s") * NC + lax.axis_index("c")
    rbase = wid * NCH
    base = wid * BPW

    pltpu.sync_copy(uid_hbm.at[pl.ds(rbase, NCH)], uid_v)
    pltpu.sync_copy(pid_hbm.at[pl.ds(rbase, NCH)], pid_v)
    pltpu.sync_copy(nid_hbm.at[pl.ds(rbase, NCH)], nid_v)

    copies = []
    for k in range(NCH):
        dst = pl.ds(k * CHUNK, CHUNK)
        copies.append(pltpu.async_copy(ut_hbm.at[uid_v.at[k]], u_rows.at[dst], sem))
        copies.append(pltpu.async_copy(pt_hbm.at[pid_v.at[k]], i_rows.at[dst], sem))
        copies.append(pltpu.async_copy(pt_hbm.at[nid_v.at[k]], j_rows.at[dst], sem))
    for cp in copies:
        cp.wait()

    lane = lax.iota(jnp.int32, L)

    def group(g, carry):
        rows = g * L + lane
        accp = jnp.zeros((L,), jnp.float32)
        accn = jnp.zeros((L,), jnp.float32)
        for c in range(D):
            cc = jnp.full((L,), c, jnp.int32)
            uc = plsc.load_gather(u_rows, [rows, cc])
            ic = plsc.load_gather(i_rows, [rows, cc])
            jc = plsc.load_gather(j_rows, [rows, cc])
            accp = accp + uc * ic
            accn = accn + uc * jc
        ps_v[pl.ds(g * L, L)] = accp
        ns_v[pl.ds(g * L, L)] = accn
        return carry

    lax.fori_loop(0, NGRP, group, 0)

    pltpu.sync_copy(ps_v, pos_out.at[pl.ds(base, BPW)])
    pltpu.sync_copy(ns_v, neg_out.at[pl.ds(base, BPW)])


@functools.cache
def _build():
    mesh = plsc.VectorSubcoreMesh(
        core_axis_name="c", subcore_axis_name="s", num_cores=NC, num_subcores=NS
    )
    return pl.kernel(
        _bpr_sc_body,
        out_type=(
            jax.ShapeDtypeStruct((B,), jnp.float32),
            jax.ShapeDtypeStruct((B,), jnp.float32),
        ),
        mesh=mesh,
        scratch_types=[
            pltpu.VMEM((NCH, CHUNK), jnp.int32),
            pltpu.VMEM((NCH, CHUNK), jnp.int32),
            pltpu.VMEM((NCH, CHUNK), jnp.int32),
            pltpu.VMEM((BPW, D), jnp.float32),
            pltpu.VMEM((BPW, D), jnp.float32),
            pltpu.VMEM((BPW, D), jnp.float32),
            pltpu.VMEM((BPW,), jnp.float32),
            pltpu.VMEM((BPW,), jnp.float32),
            pltpu.SemaphoreType.DMA,
        ],
        compiler_params=pltpu.CompilerParams(
            needs_layout_passes=False, use_tc_tiling_on_sc=False
        ),
    )


def kernel(user_ids, pos_pids, neg_pids, user_table, playlist_table):
    uid2 = user_ids.astype(jnp.int32).reshape(B // CHUNK, CHUNK)
    pid2 = pos_pids.astype(jnp.int32).reshape(B // CHUNK, CHUNK)
    nid2 = neg_pids.astype(jnp.int32).reshape(B // CHUNK, CHUNK)
    pos, neg = _build()(uid2, pid2, nid2, user_table, playlist_table)
    return (pos, neg)
